# baseline (speedup 1.0000x reference)
import jax
import jax.numpy as jnp
from jax.experimental import pallas as pl
from jax.experimental.pallas import tpu as pltpu

N, C = 16384, 1000
BLOCK = 2048


def _kl_block(s_ref, t_ref, out_ref):
    t = t_ref[...]
    s = s_ref[...]

    m1 = jnp.max(t, axis=1, keepdims=True)
    eq = t == m1
    dup = jnp.sum(eq.astype(jnp.float32), axis=1, keepdims=True) > 1.5
    m2 = jnp.max(jnp.where(eq, -jnp.inf, t), axis=1, keepdims=True)
    gap = m1 - m2
    sharp = (gap > 0.6) & (gap <= 0.8) & jnp.logical_not(dup)
    scale = jnp.where(sharp, 1.0 / 0.7, 1.0)

    ts = t * scale
    mT = m1 * scale
    eT = jnp.exp(ts - mT)
    seT = jnp.sum(eT, axis=1, keepdims=True)
    num = jnp.sum(eT * (ts - s), axis=1, keepdims=True)

    mS = jnp.max(s, axis=1, keepdims=True)
    seS = jnp.sum(jnp.exp(s - mS), axis=1, keepdims=True)

    rowsum = num / seT - mT - jnp.log(seT) + mS + jnp.log(seS)
    total = jnp.sum(rowsum) * (1.0 / N)

    @pl.when(pl.program_id(0) == 0)
    def _():
        out_ref[0, 0] = 0.0

    out_ref[0, 0] += total


@jax.jit
def kernel(preds_S, preds_T):
    out = pl.pallas_call(
        _kl_block,
        grid=(N // BLOCK,),
        in_specs=[
            pl.BlockSpec((BLOCK, C), lambda i: (i, 0)),
            pl.BlockSpec((BLOCK, C), lambda i: (i, 0)),
        ],
        out_specs=pl.BlockSpec(memory_space=pltpu.SMEM),
        out_shape=jax.ShapeDtypeStruct((1, 1), jnp.float32),
    )(preds_S, preds_T)
    return out[0, 0]
